# baseline (speedup 1.0000x reference)
import functools

import numpy as np
import jax
import jax.numpy as jnp
from jax import lax
from jax.experimental import pallas as pl
from jax.experimental.pallas import tpu as pltpu
from jax.experimental.pallas import tpu_sc as plsc

N = 256
B = 1024
K = N * (N + 1) // 2
NCHUNK = 2
CK = K // NCHUNK
NWORKERS = 32
BPW = B // NWORKERS
L = 16


def _flat_triu_indices() -> np.ndarray:
    r, c = np.triu_indices(N)
    return (r * N + c).astype(np.int32)


def _spd_body(x_hbm, idx_hbm, out_hbm, idx_v, mat_v, obuf_v, sem_in, sem_out):
    wid = lax.axis_index("s") * 2 + lax.axis_index("c")
    pltpu.sync_copy(idx_hbm, idx_v)

    def batch_body(bi, carry):
        b = wid * BPW + bi
        in_off = pl.multiple_of(b * (N * N), 8)
        pltpu.async_copy(x_hbm.at[pl.ds(in_off, N * N)], mat_v, sem_in).wait()
        for c in range(NCHUNK):
            def gather_body(t, _):
                off = t * L
                iv = idx_v[pl.ds(c * CK + off, L)]
                obuf_v[pl.ds(off, L)] = plsc.load_gather(mat_v, [iv])
                return _
            lax.fori_loop(0, CK // L, gather_body, 0)
            out_off = pl.multiple_of(b * K + c * CK, 8)
            pltpu.async_copy(obuf_v, out_hbm.at[pl.ds(out_off, CK)],
                             sem_out).wait()
        return carry

    lax.fori_loop(0, BPW, batch_body, 0)


def kernel(input):
    idx = jnp.asarray(_flat_triu_indices())
    x = input.reshape(B * N * N)
    mesh = plsc.VectorSubcoreMesh(core_axis_name="c", subcore_axis_name="s")
    spd = functools.partial(
        pl.kernel,
        mesh=mesh,
        out_type=jax.ShapeDtypeStruct((B * K,), jnp.float32),
        compiler_params=pltpu.CompilerParams(needs_layout_passes=False),
        scratch_types=[
            pltpu.VMEM((K,), jnp.int32),
            pltpu.VMEM((N * N,), jnp.float32),
            pltpu.VMEM((CK,), jnp.float32),
            pltpu.SemaphoreType.DMA,
            pltpu.SemaphoreType.DMA,
        ],
    )(_spd_body)
    return spd(x, idx).reshape(B, K)


# speedup vs baseline: 1.3673x; 1.3673x over previous
import functools

import jax
import jax.numpy as jnp
from jax import lax
from jax.experimental import pallas as pl
from jax.experimental.pallas import tpu as pltpu
from jax.experimental.pallas import tpu_sc as plsc

N = 256
B = 1024
K = N * (N + 1) // 2
NWORKERS = 32
BPW = B // NWORKERS
L = 16
R = 32
NJ = N // R
W = [N - R * j for j in range(NJ)]
COL0 = [R * j for j in range(NJ)]
OFF = [R * j * N - (R * j) * (R * j - 1) // 2 for j in range(NJ)]
PITCH = N + R
OB = K + R


def _spd_body(x_hbm, out_hbm, inbuf, obuf, sem_in0, sem_in1, sem_out0,
              sem_out1):
    wid = lax.axis_index("s") * 2 + lax.axis_index("c")
    b0 = wid * BPW
    sems_in = (sem_in0, sem_in1)
    sems_out = (sem_out0, sem_out1)

    def issue_in(b, j):
        rbase = pl.multiple_of((b0 + b) * N + R * j, 8)
        src = x_hbm.at[pl.ds(rbase, R), pl.ds(COL0[j], W[j])]
        dst = inbuf.at[j % 2, :, pl.ds(0, W[j])]
        pltpu.async_copy(src, dst, sems_in[j % 2])

    def wait_in(j):
        src = x_hbm.at[pl.ds(0, R), pl.ds(COL0[j], W[j])]
        dst = inbuf.at[j % 2, :, pl.ds(0, W[j])]
        pltpu.make_async_copy(src, dst, sems_in[j % 2]).wait()

    def out_desc(u, off):
        return pltpu.make_async_copy(
            obuf.at[u, pl.ds(0, K)], out_hbm.at[pl.ds(off, K)], sems_out[u])

    def compute_chunk(u, j):
        w = W[j]

        def row_body(li, ooff):
            for t in range(w // L):
                obuf[u, pl.ds(ooff + L * t, L)] = (
                    inbuf[j % 2, li, pl.ds(li + L * t, L)])
            return ooff + (w - li)

        lax.fori_loop(0, R, row_body, OFF[j])

    issue_in(0, 0)

    def pair_body(bp, carry):
        for u in (0, 1):
            b = bp * 2 + u

            @pl.when(bp >= 1)
            def _():
                out_desc(u, 0).wait()

            for j in range(NJ):
                wait_in(j)
                if j < NJ - 1:
                    issue_in(b, j + 1)
                else:
                    issue_in(jnp.minimum(b + 1, BPW - 1), 0)
                compute_chunk(u, j)
            off = pl.multiple_of((b0 + b) * K, 8)
            out_desc(u, off).start()
        return carry

    lax.fori_loop(0, BPW // 2, pair_body, 0)
    out_desc(0, 0).wait()
    out_desc(1, 0).wait()
    wait_in(0)


def kernel(input):
    x = input.reshape(B * N, N)
    mesh = plsc.VectorSubcoreMesh(core_axis_name="c", subcore_axis_name="s")
    spd = functools.partial(
        pl.kernel,
        mesh=mesh,
        out_type=jax.ShapeDtypeStruct((B * K,), jnp.float32),
        compiler_params=pltpu.CompilerParams(
            needs_layout_passes=False, use_tc_tiling_on_sc=False),
        scratch_types=[
            pltpu.VMEM((2, R, PITCH), jnp.float32),
            pltpu.VMEM((2, OB), jnp.float32),
            pltpu.SemaphoreType.DMA,
            pltpu.SemaphoreType.DMA,
            pltpu.SemaphoreType.DMA,
            pltpu.SemaphoreType.DMA,
        ],
    )(_spd_body)
    return spd(x).reshape(B, K)


# speedup vs baseline: 1.4451x; 1.0569x over previous
import functools

import jax
import jax.numpy as jnp
from jax import lax
from jax.experimental import pallas as pl
from jax.experimental.pallas import tpu as pltpu
from jax.experimental.pallas import tpu_sc as plsc

N = 256
B = 1024
K = N * (N + 1) // 2
NWORKERS = 32
BPW = B // NWORKERS
L = 16
R = 32
NJ = N // R
W = [N - R * j for j in range(NJ)]
OFF = [R * j * N - (R * j) * (R * j - 1) // 2 for j in range(NJ)]
CHUNK = R * N
IB = CHUNK + 2 * L
OB = K + R


def _spd_body(x_hbm, out_hbm, inbuf, obuf, sem_in0, sem_in1, sem_out0,
              sem_out1):
    wid = lax.axis_index("s") * 2 + lax.axis_index("c")
    b0 = wid * BPW
    sems_in = (sem_in0, sem_in1)
    sems_out = (sem_out0, sem_out1)

    def issue_in(b, j):
        src_off = pl.multiple_of(((b0 + b) * NJ + j) * CHUNK, 8)
        src = x_hbm.at[pl.ds(src_off, CHUNK)]
        dst = inbuf.at[j % 2, pl.ds(0, CHUNK)]
        pltpu.async_copy(src, dst, sems_in[j % 2])

    def wait_in(j):
        src = x_hbm.at[pl.ds(0, CHUNK)]
        dst = inbuf.at[j % 2, pl.ds(0, CHUNK)]
        pltpu.make_async_copy(src, dst, sems_in[j % 2]).wait()

    def out_desc(u, off):
        return pltpu.make_async_copy(
            obuf.at[u, pl.ds(0, K)], out_hbm.at[pl.ds(off, K)], sems_out[u])

    def compute_chunk(u, j):
        w = W[j]

        def row_body(li, ooff):
            sbase = li * (N + 1) + R * j
            vs = [inbuf[j % 2, pl.ds(sbase + L * t, L)] for t in range(w // L)]
            for t, v in enumerate(vs):
                obuf[u, pl.ds(ooff + L * t, L)] = v
            return ooff + (w - li)

        lax.fori_loop(0, R, row_body, OFF[j])

    issue_in(0, 0)

    def pair_body(bp, carry):
        for u in (0, 1):
            b = bp * 2 + u

            @pl.when(bp >= 1)
            def _():
                out_desc(u, 0).wait()

            for j in range(NJ):
                wait_in(j)
                if j < NJ - 1:
                    issue_in(b, j + 1)
                else:
                    issue_in(jnp.minimum(b + 1, BPW - 1), 0)
                compute_chunk(u, j)
            off = pl.multiple_of((b0 + b) * K, 8)
            out_desc(u, off).start()
        return carry

    lax.fori_loop(0, BPW // 2, pair_body, 0)
    out_desc(0, 0).wait()
    out_desc(1, 0).wait()
    wait_in(0)


def kernel(input):
    x = input.reshape(B * N * N)
    mesh = plsc.VectorSubcoreMesh(core_axis_name="c", subcore_axis_name="s")
    spd = functools.partial(
        pl.kernel,
        mesh=mesh,
        out_type=jax.ShapeDtypeStruct((B * K,), jnp.float32),
        compiler_params=pltpu.CompilerParams(
            needs_layout_passes=False, use_tc_tiling_on_sc=False),
        scratch_types=[
            pltpu.VMEM((2, IB), jnp.float32),
            pltpu.VMEM((2, OB), jnp.float32),
            pltpu.SemaphoreType.DMA,
            pltpu.SemaphoreType.DMA,
            pltpu.SemaphoreType.DMA,
            pltpu.SemaphoreType.DMA,
        ],
    )(_spd_body)
    return spd(x).reshape(B, K)


# speedup vs baseline: 2.0504x; 1.4189x over previous
import functools

import jax
import jax.numpy as jnp
from jax import lax
from jax.experimental import pallas as pl
from jax.experimental.pallas import tpu as pltpu
from jax.experimental.pallas import tpu_sc as plsc

N = 256
B = 1024
K = N * (N + 1) // 2
NWORKERS = 32
BPW = B // NWORKERS
L = 16
R = 32
NJ = N // R
RT = R // 8
W = [N - R * j for j in range(NJ)]
OFF = [R * j * N - (R * j) * (R * j - 1) // 2 for j in range(NJ)]
OB = K + R


def _spd_body(x_hbm, out_hbm, inbuf, obuf0, obuf1, sem_in0, sem_in1,
              sem_out0, sem_out1):
    wid = lax.axis_index("s") * 2 + lax.axis_index("c")
    b0 = wid * BPW
    sems_in = (sem_in0, sem_in1)
    sems_out = (sem_out0, sem_out1)
    obufs = (obuf0, obuf1)
    iota = lax.iota(jnp.int32, L)

    def issue_in(b, j):
        src = x_hbm.at[b0 + b, pl.ds(RT * j, RT)]
        pltpu.async_copy(src, inbuf.at[j % 2], sems_in[j % 2])

    def wait_in(j):
        src = x_hbm.at[0, pl.ds(RT * j, RT)]
        pltpu.make_async_copy(src, inbuf.at[j % 2], sems_in[j % 2]).wait()

    def out_desc(u, off):
        return pltpu.make_async_copy(
            obufs[u].at[pl.ds(0, K)], out_hbm.at[pl.ds(off, K)], sems_out[u])

    def compute_chunk(u, j):
        w = W[j]
        ref3 = inbuf.at[j % 2]
        obuf = obufs[u]

        def row_body(li, ooff):
            rt = jnp.broadcast_to(li // 8, (L,)).astype(jnp.int32)
            r = jnp.broadcast_to(li % 8, (L,)).astype(jnp.int32)
            cvec = R * j + li + iota
            vs = [plsc.load_gather(ref3, [rt, r, cvec + L * t])
                  for t in range(w // L)]
            for t, v in enumerate(vs):
                obuf[pl.ds(ooff + L * t, L)] = v
            return ooff + (w - li)

        lax.fori_loop(0, R, row_body, OFF[j])

    issue_in(0, 0)

    def pair_body(bp, carry):
        for u in (0, 1):
            b = bp * 2 + u

            @pl.when(bp >= 1)
            def _():
                out_desc(u, 0).wait()

            for j in range(NJ):
                wait_in(j)
                if j < NJ - 1:
                    issue_in(b, j + 1)
                else:
                    issue_in(jnp.minimum(b + 1, BPW - 1), 0)
                compute_chunk(u, j)
            off = pl.multiple_of((b0 + b) * K, 8)
            out_desc(u, off).start()
        return carry

    lax.fori_loop(0, BPW // 2, pair_body, 0)
    out_desc(0, 0).wait()
    out_desc(1, 0).wait()
    wait_in(0)


def kernel(input):
    x = input.reshape(B, N // 8, 8, N)
    mesh = plsc.VectorSubcoreMesh(core_axis_name="c", subcore_axis_name="s")
    spd = functools.partial(
        pl.kernel,
        mesh=mesh,
        out_type=jax.ShapeDtypeStruct((B * K,), jnp.float32),
        compiler_params=pltpu.CompilerParams(needs_layout_passes=False),
        scratch_types=[
            pltpu.VMEM((2, RT, 8, N), jnp.float32),
            pltpu.VMEM((OB,), jnp.float32),
            pltpu.VMEM((OB,), jnp.float32),
            pltpu.SemaphoreType.DMA,
            pltpu.SemaphoreType.DMA,
            pltpu.SemaphoreType.DMA,
            pltpu.SemaphoreType.DMA,
        ],
    )(_spd_body)
    return spd(x).reshape(B, K)


# speedup vs baseline: 2.1651x; 1.0560x over previous
import functools

import jax
import jax.numpy as jnp
from jax import lax
from jax.experimental import pallas as pl
from jax.experimental.pallas import tpu as pltpu
from jax.experimental.pallas import tpu_sc as plsc

N = 256
B = 1024
K = N * (N + 1) // 2
NWORKERS = 32
BPW = B // NWORKERS
L = 16
R = 32
NJ = N // R
RT = R // 8
W = [N - R * j for j in range(NJ)]
OFF = [R * j * N - (R * j) * (R * j - 1) // 2 for j in range(NJ)]
OB = K + R


def _spd_body(x_hbm, out_hbm, inbuf, obuf0, obuf1, sem_in0, sem_in1,
              sem_out0, sem_out1):
    wid = lax.axis_index("s") * 2 + lax.axis_index("c")
    b0 = wid * BPW
    sems_in = (sem_in0, sem_in1)
    sems_out = (sem_out0, sem_out1)
    obufs = (obuf0, obuf1)
    iota = lax.iota(jnp.int32, L)

    def issue_in(b, j):
        c0 = 128 if R * j >= 128 else 0
        src = x_hbm.at[b0 + b, pl.ds(RT * j, RT), :, pl.ds(c0, N - c0)]
        dst = inbuf.at[j % 2, :, :, pl.ds(c0, N - c0)]
        pltpu.async_copy(src, dst, sems_in[j % 2])

    def wait_in(j):
        c0 = 128 if R * j >= 128 else 0
        src = x_hbm.at[0, pl.ds(RT * j, RT), :, pl.ds(c0, N - c0)]
        dst = inbuf.at[j % 2, :, :, pl.ds(c0, N - c0)]
        pltpu.make_async_copy(src, dst, sems_in[j % 2]).wait()

    def out_desc(u, off):
        return pltpu.make_async_copy(
            obufs[u].at[pl.ds(0, K)], out_hbm.at[pl.ds(off, K)], sems_out[u])

    def compute_chunk(u, j):
        w = W[j]
        ref3 = inbuf.at[j % 2]
        obuf = obufs[u]

        def row_body(li, ooff):
            rt = jnp.broadcast_to(li // 8, (L,)).astype(jnp.int32)
            r = jnp.broadcast_to(li % 8, (L,)).astype(jnp.int32)
            cvec = R * j + li + iota
            vs = [plsc.load_gather(ref3, [rt, r, cvec + L * t])
                  for t in range(w // L)]
            for t, v in enumerate(vs):
                obuf[pl.ds(ooff + L * t, L)] = v
            return ooff + (w - li)

        lax.fori_loop(0, R, row_body, OFF[j])

    issue_in(0, 0)

    def pair_body(bp, carry):
        for u in (0, 1):
            b = bp * 2 + u

            @pl.when(bp >= 1)
            def _():
                out_desc(u, 0).wait()

            for j in range(NJ):
                wait_in(j)
                if j < NJ - 1:
                    issue_in(b, j + 1)
                else:
                    issue_in(jnp.minimum(b + 1, BPW - 1), 0)
                compute_chunk(u, j)
            off = pl.multiple_of((b0 + b) * K, 8)
            out_desc(u, off).start()
        return carry

    lax.fori_loop(0, BPW // 2, pair_body, 0)
    out_desc(0, 0).wait()
    out_desc(1, 0).wait()
    wait_in(0)


def kernel(input):
    x = input.reshape(B, N // 8, 8, N)
    mesh = plsc.VectorSubcoreMesh(core_axis_name="c", subcore_axis_name="s")
    spd = functools.partial(
        pl.kernel,
        mesh=mesh,
        out_type=jax.ShapeDtypeStruct((B * K,), jnp.float32),
        compiler_params=pltpu.CompilerParams(needs_layout_passes=False),
        scratch_types=[
            pltpu.VMEM((2, RT, 8, N), jnp.float32),
            pltpu.VMEM((OB,), jnp.float32),
            pltpu.VMEM((OB,), jnp.float32),
            pltpu.SemaphoreType.DMA,
            pltpu.SemaphoreType.DMA,
            pltpu.SemaphoreType.DMA,
            pltpu.SemaphoreType.DMA,
        ],
    )(_spd_body)
    return spd(x).reshape(B, K)


# speedup vs baseline: 2.7043x; 1.2490x over previous
import functools

import jax
import jax.numpy as jnp
from jax import lax
from jax.experimental import pallas as pl
from jax.experimental.pallas import tpu as pltpu
from jax.experimental.pallas import tpu_sc as plsc

N = 256
B = 1024
K = N * (N + 1) // 2
NWORKERS = 32
BPW = B // NWORKERS
L = 16
R = 64
NJ = N // R
RT = R // 8
W = [N - R * j for j in range(NJ)]
OFF = [R * j * N - (R * j) * (R * j - 1) // 2 for j in range(NJ)]
OB = K + R
assert all(w % L == 0 for w in W)


def _spd_body(x_hbm, out_hbm, inbuf, obuf0, obuf1, sem_in0, sem_in1,
              sem_out0, sem_out1):
    wid = lax.axis_index("s") * 2 + lax.axis_index("c")
    b0 = wid * BPW
    sems_in = (sem_in0, sem_in1)
    sems_out = (sem_out0, sem_out1)
    obufs = (obuf0, obuf1)
    iota = lax.iota(jnp.int32, L)

    def issue_in(b, j):
        c0 = 128 if R * j >= 128 else 0
        src = x_hbm.at[b0 + b, pl.ds(RT * j, RT), :, pl.ds(c0, N - c0)]
        dst = inbuf.at[j % 2, :, :, pl.ds(c0, N - c0)]
        pltpu.async_copy(src, dst, sems_in[j % 2])

    def wait_in(j):
        c0 = 128 if R * j >= 128 else 0
        src = x_hbm.at[0, pl.ds(RT * j, RT), :, pl.ds(c0, N - c0)]
        dst = inbuf.at[j % 2, :, :, pl.ds(c0, N - c0)]
        pltpu.make_async_copy(src, dst, sems_in[j % 2]).wait()

    def out_desc(u, off):
        return pltpu.make_async_copy(
            obufs[u].at[pl.ds(0, K)], out_hbm.at[pl.ds(off, K)], sems_out[u])

    def compute_chunk(u, j):
        w = W[j]
        ref3 = inbuf.at[j % 2]
        obuf = obufs[u]

        def row_pair_body(p, ooff):
            li = 2 * p
            vs = []
            for s in (0, 1):
                lis = li + s
                rt = jnp.broadcast_to(lis // 8, (L,)).astype(jnp.int32)
                r = jnp.broadcast_to(lis % 8, (L,)).astype(jnp.int32)
                cvec = R * j + lis + iota
                vs.append([plsc.load_gather(ref3, [rt, r, cvec + L * t])
                           for t in range(w // L)])
            o1 = ooff + (w - li)
            for t, v in enumerate(vs[0]):
                obuf[pl.ds(ooff + L * t, L)] = v
            for t, v in enumerate(vs[1]):
                obuf[pl.ds(o1 + L * t, L)] = v
            return o1 + (w - li - 1)

        lax.fori_loop(0, R // 2, row_pair_body, OFF[j])

    issue_in(0, 0)

    def pair_body(bp, carry):
        for u in (0, 1):
            b = bp * 2 + u

            @pl.when(bp >= 1)
            def _():
                out_desc(u, 0).wait()

            for j in range(NJ):
                wait_in(j)
                if j < NJ - 1:
                    issue_in(b, j + 1)
                else:
                    issue_in(jnp.minimum(b + 1, BPW - 1), 0)
                compute_chunk(u, j)
            off = pl.multiple_of((b0 + b) * K, 8)
            out_desc(u, off).start()
        return carry

    lax.fori_loop(0, BPW // 2, pair_body, 0)
    out_desc(0, 0).wait()
    out_desc(1, 0).wait()
    wait_in(0)


def kernel(input):
    x = input.reshape(B, N // 8, 8, N)
    mesh = plsc.VectorSubcoreMesh(core_axis_name="c", subcore_axis_name="s")
    spd = functools.partial(
        pl.kernel,
        mesh=mesh,
        out_type=jax.ShapeDtypeStruct((B * K,), jnp.float32),
        compiler_params=pltpu.CompilerParams(needs_layout_passes=False),
        scratch_types=[
            pltpu.VMEM((2, RT, 8, N), jnp.float32),
            pltpu.VMEM((OB,), jnp.float32),
            pltpu.VMEM((OB,), jnp.float32),
            pltpu.SemaphoreType.DMA,
            pltpu.SemaphoreType.DMA,
            pltpu.SemaphoreType.DMA,
            pltpu.SemaphoreType.DMA,
        ],
    )(_spd_body)
    return spd(x).reshape(B, K)
